# baseline (speedup 1.0000x reference)
import jax
import jax.numpy as jnp
from jax.experimental import pallas as pl
from jax.experimental.pallas import tpu as pltpu

_B, _C, _H, _W = 4, 16, 224, 224
_NE = 6
_LC = 0.01
_RC = 16
_NCH = _H // _RC
_PH = _H + 32
_PW = _W + 2


def _leaky(x):
    return jnp.where(x >= 0, x, _LC * x)


def _pool_sm(x):
    x4 = x.reshape(_C, 14, 16, _W)
    s1 = jnp.sum(x4, axis=2)
    m1 = jnp.max(x4, axis=2)
    s2 = jnp.swapaxes(jnp.sum(jnp.swapaxes(s1, 1, 2).reshape(_C, 14, 16, 14),
                              axis=2), 1, 2)
    m2 = jnp.swapaxes(jnp.max(jnp.swapaxes(m1, 1, 2).reshape(_C, 14, 16, 14),
                              axis=2), 1, 2)
    return s2 * (1.0 / 256.0) + m2


def _fill_padded(dst_ref, x):
    dst_ref[:, 0:16, :] = jnp.zeros((_C, 16, _PW), jnp.bfloat16)
    dst_ref[:, _PH - 16:_PH, :] = jnp.zeros((_C, 16, _PW), jnp.bfloat16)
    zc = jnp.zeros((_C, _H, 1), jnp.bfloat16)
    dst_ref[:, 16:16 + _H, :] = jnp.concatenate(
        [zc, x.astype(jnp.bfloat16), zc], axis=2)


def _dystack(src_ref, r0, nc):
    xs = src_ref[:, pl.ds(r0, _RC + 32), :]
    rows = [jax.lax.slice(xs, (0, 15 + dy, 0), (nc, 15 + dy + _RC, _PW))
            for dy in range(3)]
    return jnp.stack(rows, axis=0).reshape(3 * nc, _RC, _PW)


def _conv_terms(w_ref, x3, m, terms):
    out = jax.lax.dot_general(w_ref[...], x3, (((1,), (0,)), ((), ())),
                              preferred_element_type=jnp.float32)
    acc = None
    for (ro, dx) in terms:
        t = jax.lax.slice(out, (ro, 0, dx), (ro + m, _RC, dx + _W))
        acc = t if acc is None else acc + t
    return acc


_T1 = [(h * 48 + dx * 16, dx) for h in range(2) for dx in range(3)]
_TS = [(dx * 32, dx) for dx in range(3)]


def _conv_layer(src_ref, w_ref, bias_ref, dst_ref):
    dst_ref[:, 0:16, :] = jnp.zeros((2 * _C, 16, _PW), jnp.bfloat16)
    dst_ref[:, _PH - 16:_PH, :] = jnp.zeros((2 * _C, 16, _PW), jnp.bfloat16)

    def chunk(k, carry):
        r0 = k * _RC
        out = _conv_terms(w_ref, _dystack(src_ref, r0, _C), _C, _T1)
        out = _leaky(out + bias_ref[...])
        hi = out.astype(jnp.bfloat16)
        lo = (out - hi.astype(jnp.float32)).astype(jnp.bfloat16)
        zc = jnp.zeros((2 * _C, _RC, 1), jnp.bfloat16)
        dst_ref[:, pl.ds(r0 + 16, _RC), :] = jnp.concatenate(
            [zc, jnp.concatenate([hi, lo], axis=0), zc], axis=2)
        return carry
    jax.lax.fori_loop(0, _NCH, chunk, 0, unroll=False)


def _conv_out_accum(src_ref, w_ref, bias_ref, gs, first, y_ref):
    def chunk(k, carry):
        r0 = k * _RC
        out = _conv_terms(w_ref, _dystack(src_ref, r0, 2 * _C), _C, _T2)
        contrib = gs * _leaky(out + bias_ref[...])
        if first:
            y_ref[0, :, pl.ds(r0, _RC), :] = contrib
        else:
            y_ref[0, :, pl.ds(r0, _RC), :] += contrib
        return carry
    jax.lax.fori_loop(0, _NCH, chunk, 0, unroll=False)


_T2 = [(h * 48 + dx * 16, dx) for h in range(2) for dx in range(3)]


def _sobel_accum(src_ref, wsob_ref, gs, y_ref):
    def chunk(k, carry):
        r0 = k * _RC
        out = _conv_terms(wsob_ref, _dystack(src_ref, r0, _C), 2 * _C, _TS)
        contrib = gs * (jnp.abs(out[:_C]) + jnp.abs(out[_C:]))
        y_ref[0, :, pl.ds(r0, _RC), :] += contrib
        return carry
    jax.lax.fori_loop(0, _NCH, chunk, 0, unroll=False)


def _pool_body(v_ref, i_ref, s_ref):
    s_ref[0, :_C] = _leaky(_pool_sm(v_ref[0]))
    s_ref[0, _C:] = _leaky(_pool_sm(i_ref[0]))


def _fused_body(vh_ref, vl_ref, ihh_ref, il_ref, v_ref, i_ref, g_ref,
                w1a_ref, b1a_ref, w2a_ref, b2a_ref,
                w1b_ref, b1b_ref, w2b_ref, b2b_ref,
                w1c_ref, b1c_ref, w2c_ref, b2c_ref,
                w1d_ref, b1d_ref, w2d_ref, b2d_ref,
                wsob_ref,
                y_ref, pa_ref, pb_ref):
    g = g_ref[0, 0]

    conv_sets = [
        (vh_ref, w1a_ref, b1a_ref, w2a_ref, b2a_ref),
        (vl_ref, w1b_ref, b1b_ref, w2b_ref, b2b_ref),
        (ihh_ref, w1c_ref, b1c_ref, w2c_ref, b2c_ref),
        (il_ref, w1d_ref, b1d_ref, w2d_ref, b2d_ref),
    ]
    for e, (x_ref, w1, b1, w2, b2) in enumerate(conv_sets):
        _fill_padded(pa_ref, x_ref[0])
        _conv_layer(pa_ref, w1, b1, pb_ref)
        _conv_out_accum(pb_ref, w2, b2, g[e], e == 0, y_ref)

    _fill_padded(pa_ref, v_ref[0])
    _sobel_accum(pa_ref, wsob_ref, g[4], y_ref)
    _fill_padded(pa_ref, i_ref[0])
    _sobel_accum(pa_ref, wsob_ref, g[5], y_ref)


def _hilo_rows(w):
    hi = w.astype(jnp.bfloat16)
    lo = (w - hi.astype(jnp.float32)).astype(jnp.bfloat16)
    return jnp.concatenate([hi, lo], axis=0)


def _wprep1(w):
    w48 = jnp.transpose(w, (3, 0, 2, 1)).reshape(48, 48)
    return _hilo_rows(w48)


def _wprep2(w):
    w48 = jnp.transpose(w, (3, 0, 2, 1)).reshape(48, 3, _C)
    w96 = jnp.concatenate([w48, w48], axis=2).reshape(48, 96)
    return _hilo_rows(w96)


def kernel(vis_h, vis_l, ir_h, ir_l, vis, ir, params):
    p = params
    noise = jax.random.normal(jax.random.key(1), (_B, _NE), dtype=jnp.float32)

    img = pl.BlockSpec((1, _C, _H, _W), lambda b: (b, 0, 0, 0))
    s_local = pl.pallas_call(
        _pool_body,
        grid=(_B,),
        in_specs=[img, img],
        out_specs=pl.BlockSpec((1, 2 * _C, 14, 14), lambda b: (b, 0, 0, 0)),
        out_shape=jax.ShapeDtypeStruct((_B, 2 * _C, 14, 14), jnp.float32),
    )(vis, ir)

    s_flat = s_local.reshape(_B, 2 * _C * 14 * 14)
    clean_logits = s_flat @ p['w_gate']
    noise_stddev = jax.nn.softplus(s_flat @ p['w_noise']) + _LC
    logits = clean_logits + noise * noise_stddev
    g = jax.nn.softmax(logits, axis=1)
    importance = g.sum(axis=0)
    loss = (jnp.var(importance, ddof=1) / (importance.mean() ** 2 + 1e-10)) * _LC

    eye = jnp.eye(_C, dtype=jnp.float32)
    cwx = p['sobel_vis']['wx'][:, 0]
    cwy = p['sobel_vis']['wy'][:, 0]
    cw = jnp.stack([cwx, cwy], axis=0)
    t = (jnp.transpose(cw, (3, 0, 1, 2))[:, :, :, :, None]
         * eye[None, None, :, None, :])
    wsob = t.reshape(96, 48).astype(jnp.bfloat16)

    full = lambda a: pl.BlockSpec(a.shape, lambda b: (0,) * a.ndim)

    vis_h = vis_h.astype(jnp.bfloat16)
    vis_l = vis_l.astype(jnp.bfloat16)
    ir_h = ir_h.astype(jnp.bfloat16)
    ir_l = ir_l.astype(jnp.bfloat16)
    vis_b = vis.astype(jnp.bfloat16)
    ir_b = ir.astype(jnp.bfloat16)

    exp_args = []
    exp_specs = []
    for nm in ('exp_vis_h', 'exp_vis_l', 'exp_ir_h', 'exp_ir_l'):
        w1 = _wprep1(p[nm]['w1'])
        b1 = p[nm]['b1'].reshape(_C, 1, 1)
        w2 = _wprep2(p[nm]['w2'])
        b2 = p[nm]['b2'].reshape(_C, 1, 1)
        exp_args += [w1, b1, w2, b2]
        exp_specs += [full(w1), full(b1), full(w2), full(b2)]

    args = [vis_h, vis_l, ir_h, ir_l, vis_b, ir_b,
            g.reshape(_B, 1, _NE), *exp_args, wsob]
    specs = ([img] * 6
             + [pl.BlockSpec((1, 1, _NE), lambda b: (b, 0, 0))]
             + exp_specs + [full(wsob)])

    y = pl.pallas_call(
        _fused_body,
        grid=(_B,),
        in_specs=specs,
        out_specs=img,
        out_shape=jax.ShapeDtypeStruct((_B, _C, _H, _W), jnp.float32),
        scratch_shapes=[pltpu.VMEM((_C, _PH, _PW), jnp.bfloat16),
                        pltpu.VMEM((2 * _C, _PH, _PW), jnp.bfloat16)],
    )(*args)
    return y, loss


# speedup vs baseline: 1.4802x; 1.4802x over previous
import jax
import jax.numpy as jnp
from jax.experimental import pallas as pl
from jax.experimental.pallas import tpu as pltpu

_B, _C, _H, _W = 4, 16, 224, 224
_NE = 6
_LC = 0.01
_RC = 32
_NCH = _H // _RC
_PH = _H + 32
_PW = _W + 2


def _leaky(x):
    return jnp.where(x >= 0, x, _LC * x)


def _pool_sm(x):
    x4 = x.reshape(_C, 14, 16, _W)
    s1 = jnp.sum(x4, axis=2)
    m1 = jnp.max(x4, axis=2)
    s2 = jnp.swapaxes(jnp.sum(jnp.swapaxes(s1, 1, 2).reshape(_C, 14, 16, 14),
                              axis=2), 1, 2)
    m2 = jnp.swapaxes(jnp.max(jnp.swapaxes(m1, 1, 2).reshape(_C, 14, 16, 14),
                              axis=2), 1, 2)
    return s2 * (1.0 / 256.0) + m2


def _fill_padded(dst_ref, x):
    dst_ref[:, 0:16, :] = jnp.zeros((_C, 16, _PW), jnp.bfloat16)
    dst_ref[:, _PH - 16:_PH, :] = jnp.zeros((_C, 16, _PW), jnp.bfloat16)
    zc = jnp.zeros((_C, _H, 1), jnp.bfloat16)
    dst_ref[:, 16:16 + _H, :] = jnp.concatenate(
        [zc, x.astype(jnp.bfloat16), zc], axis=2)


def _dystack(src_ref, r0, nc):
    xs = src_ref[:, pl.ds(r0, _RC + 32), :]
    rows = [jax.lax.slice(xs, (0, 15 + dy, 0), (nc, 15 + dy + _RC, _PW))
            for dy in range(3)]
    return jnp.stack(rows, axis=0).reshape(3 * nc, _RC, _PW)


def _conv_terms(w_ref, x3, m, terms):
    out = jax.lax.dot_general(w_ref[...], x3, (((1,), (0,)), ((), ())),
                              preferred_element_type=jnp.float32)
    acc = None
    for (ro, dx) in terms:
        t = jax.lax.slice(out, (ro, 0, dx), (ro + m, _RC, dx + _W))
        acc = t if acc is None else acc + t
    return acc


_T1 = [(dx * 16, dx) for dx in range(3)]
_TS = [(dx * 32, dx) for dx in range(3)]


def _conv_layer(src_ref, w_ref, bias_ref, dst_ref):
    dst_ref[:, 0:16, :] = jnp.zeros((_C, 16, _PW), jnp.bfloat16)
    dst_ref[:, _PH - 16:_PH, :] = jnp.zeros((_C, 16, _PW), jnp.bfloat16)

    def chunk(k, carry):
        r0 = k * _RC
        out = _conv_terms(w_ref, _dystack(src_ref, r0, _C), _C, _T1)
        out = _leaky(out + bias_ref[...])
        zc = jnp.zeros((_C, _RC, 1), jnp.bfloat16)
        dst_ref[:, pl.ds(r0 + 16, _RC), :] = jnp.concatenate(
            [zc, out.astype(jnp.bfloat16), zc], axis=2)
        return carry
    jax.lax.fori_loop(0, _NCH, chunk, 0, unroll=False)


def _conv_out_accum(src_ref, w_ref, bias_ref, gs, first, y_ref):
    def chunk(k, carry):
        r0 = k * _RC
        out = _conv_terms(w_ref, _dystack(src_ref, r0, _C), _C, _T1)
        contrib = gs * _leaky(out + bias_ref[...])
        if first:
            y_ref[0, :, pl.ds(r0, _RC), :] = contrib
        else:
            y_ref[0, :, pl.ds(r0, _RC), :] += contrib
        return carry
    jax.lax.fori_loop(0, _NCH, chunk, 0, unroll=False)


def _sobel_accum(src_ref, wsob_ref, gs, y_ref):
    def chunk(k, carry):
        r0 = k * _RC
        out = _conv_terms(wsob_ref, _dystack(src_ref, r0, _C), 2 * _C, _TS)
        contrib = gs * (jnp.abs(out[:_C]) + jnp.abs(out[_C:]))
        y_ref[0, :, pl.ds(r0, _RC), :] += contrib
        return carry
    jax.lax.fori_loop(0, _NCH, chunk, 0, unroll=False)


def _pool_body(v_ref, i_ref, s_ref):
    s_ref[0, :_C] = _leaky(_pool_sm(v_ref[0]))
    s_ref[0, _C:] = _leaky(_pool_sm(i_ref[0]))


def _fused_body(vh_ref, vl_ref, ihh_ref, il_ref, v_ref, i_ref, g_ref,
                w1a_ref, b1a_ref, w2a_ref, b2a_ref,
                w1b_ref, b1b_ref, w2b_ref, b2b_ref,
                w1c_ref, b1c_ref, w2c_ref, b2c_ref,
                w1d_ref, b1d_ref, w2d_ref, b2d_ref,
                wsob_ref,
                y_ref, pa_ref, pb_ref):
    g = g_ref[0, 0]

    conv_sets = [
        (vh_ref, w1a_ref, b1a_ref, w2a_ref, b2a_ref),
        (vl_ref, w1b_ref, b1b_ref, w2b_ref, b2b_ref),
        (ihh_ref, w1c_ref, b1c_ref, w2c_ref, b2c_ref),
        (il_ref, w1d_ref, b1d_ref, w2d_ref, b2d_ref),
    ]
    for e, (x_ref, w1, b1, w2, b2) in enumerate(conv_sets):
        _fill_padded(pa_ref, x_ref[0])
        _conv_layer(pa_ref, w1, b1, pb_ref)
        _conv_out_accum(pb_ref, w2, b2, g[e], e == 0, y_ref)

    _fill_padded(pa_ref, v_ref[0])
    _sobel_accum(pa_ref, wsob_ref, g[4], y_ref)
    _fill_padded(pa_ref, i_ref[0])
    _sobel_accum(pa_ref, wsob_ref, g[5], y_ref)


def _wprep(w):
    return jnp.transpose(w, (3, 0, 2, 1)).reshape(48, 48).astype(jnp.bfloat16)


def kernel(vis_h, vis_l, ir_h, ir_l, vis, ir, params):
    p = params
    noise = jax.random.normal(jax.random.key(1), (_B, _NE), dtype=jnp.float32)

    img = pl.BlockSpec((1, _C, _H, _W), lambda b: (b, 0, 0, 0))
    s_local = pl.pallas_call(
        _pool_body,
        grid=(_B,),
        in_specs=[img, img],
        out_specs=pl.BlockSpec((1, 2 * _C, 14, 14), lambda b: (b, 0, 0, 0)),
        out_shape=jax.ShapeDtypeStruct((_B, 2 * _C, 14, 14), jnp.float32),
    )(vis, ir)

    s_flat = s_local.reshape(_B, 2 * _C * 14 * 14)
    clean_logits = s_flat @ p['w_gate']
    noise_stddev = jax.nn.softplus(s_flat @ p['w_noise']) + _LC
    logits = clean_logits + noise * noise_stddev
    g = jax.nn.softmax(logits, axis=1)
    importance = g.sum(axis=0)
    loss = (jnp.var(importance, ddof=1) / (importance.mean() ** 2 + 1e-10)) * _LC

    eye = jnp.eye(_C, dtype=jnp.float32)
    cwx = p['sobel_vis']['wx'][:, 0]
    cwy = p['sobel_vis']['wy'][:, 0]
    cw = jnp.stack([cwx, cwy], axis=0)
    t = (jnp.transpose(cw, (3, 0, 1, 2))[:, :, :, :, None]
         * eye[None, None, :, None, :])
    wsob = t.reshape(96, 48).astype(jnp.bfloat16)

    full = lambda a: pl.BlockSpec(a.shape, lambda b: (0,) * a.ndim)

    vis_h = vis_h.astype(jnp.bfloat16)
    vis_l = vis_l.astype(jnp.bfloat16)
    ir_h = ir_h.astype(jnp.bfloat16)
    ir_l = ir_l.astype(jnp.bfloat16)
    vis_b = vis.astype(jnp.bfloat16)
    ir_b = ir.astype(jnp.bfloat16)

    exp_args = []
    exp_specs = []
    for nm in ('exp_vis_h', 'exp_vis_l', 'exp_ir_h', 'exp_ir_l'):
        w1 = _wprep(p[nm]['w1'])
        b1 = p[nm]['b1'].reshape(_C, 1, 1)
        w2 = _wprep(p[nm]['w2'])
        b2 = p[nm]['b2'].reshape(_C, 1, 1)
        exp_args += [w1, b1, w2, b2]
        exp_specs += [full(w1), full(b1), full(w2), full(b2)]

    args = [vis_h, vis_l, ir_h, ir_l, vis_b, ir_b,
            g.reshape(_B, 1, _NE), *exp_args, wsob]
    specs = ([img] * 6
             + [pl.BlockSpec((1, 1, _NE), lambda b: (b, 0, 0))]
             + exp_specs + [full(wsob)])

    y = pl.pallas_call(
        _fused_body,
        grid=(_B,),
        in_specs=specs,
        out_specs=img,
        out_shape=jax.ShapeDtypeStruct((_B, _C, _H, _W), jnp.float32),
        scratch_shapes=[pltpu.VMEM((_C, _PH, _PW), jnp.bfloat16),
                        pltpu.VMEM((_C, _PH, _PW), jnp.bfloat16)],
    )(*args)
    return y, loss
